# baseline (speedup 1.0000x reference)
import functools

import jax
import jax.numpy as jnp
from jax import lax
from jax.experimental import pallas as pl
from jax.experimental.pallas import tpu as pltpu
from jax.experimental.pallas import tpu_sc as plsc

C = 16
DI = 1024
DH = 4096
DO = 1024
N = 4096
BS = 128
TH = 1024
K = DH // TH
G = C * K

_NW = 32
_CH = 64

def _sc_mesh():
    return plsc.VectorSubcoreMesh(core_axis_name="c", subcore_axis_name="s")


def _sc_gather(table, idx):
    v, d = table.shape
    bn = idx.shape[0]
    per_w = bn // _NW
    nch = per_w // _CH

    @functools.partial(
        pl.kernel, mesh=_sc_mesh(),
        out_type=jax.ShapeDtypeStruct((bn, d), table.dtype),
        scratch_types=[pltpu.VMEM((_CH,), jnp.int32),
                       pltpu.VMEM((_CH, d), table.dtype),
                       pltpu.SemaphoreType.DMA])
    def k(table_hbm, idx_hbm, out_hbm, idx_v, rows_v, sem):
        wid = lax.axis_index("s") * 2 + lax.axis_index("c")

        @pl.loop(0, nch)
        def _(i):
            base = wid * per_w + i * _CH
            pltpu.sync_copy(idx_hbm.at[pl.ds(base, _CH)], idx_v)
            pltpu.async_copy(table_hbm.at[idx_v], rows_v, sem).wait()
            pltpu.sync_copy(rows_v, out_hbm.at[pl.ds(base, _CH)])

    return k(table, idx)


def _sc_scatter(rows, idx, v):
    bn, d = rows.shape
    per_w = bn // _NW
    nch = per_w // _CH

    @functools.partial(
        pl.kernel, mesh=_sc_mesh(),
        out_type=jax.ShapeDtypeStruct((v, d), rows.dtype),
        scratch_types=[pltpu.VMEM((_CH,), jnp.int32),
                       pltpu.VMEM((_CH, d), rows.dtype),
                       pltpu.SemaphoreType.DMA])
    def k(rows_hbm, idx_hbm, out_hbm, idx_v, rows_v, sem):
        wid = lax.axis_index("s") * 2 + lax.axis_index("c")

        @pl.loop(0, nch)
        def _(i):
            base = wid * per_w + i * _CH
            pltpu.sync_copy(idx_hbm.at[pl.ds(base, _CH)], idx_v)
            pltpu.sync_copy(rows_hbm.at[pl.ds(base, _CH)], rows_v)
            pltpu.async_copy(rows_v, out_hbm.at[idx_v], sem).wait()

    return k(rows, idx)


def _mlp_body(ts_r, te_r,
              x_ref, w1_ref, b1_ref, w2_ref, b2_ref, out_ref):
    g = pl.program_id(0)
    c = g // K
    k = g - c * K

    @pl.when(g == 0)
    def _init():
        out_ref[...] = jnp.zeros_like(out_ref)

    s = ts_r[c]
    e = te_r[c]

    @pl.when(e > s)
    def _work():
        a0 = (s // 8) * 8
        nsub = (e - a0 + BS - 1) // BS

        def sub(j, _):
            r0l = a0 + j * BS
            r0 = jnp.minimum(r0l, N - BS)
            rows = r0 + lax.broadcasted_iota(jnp.int32, (BS, 1), 0)
            gate = ((rows >= jnp.maximum(s, r0l)) & (rows < e)
                    ).astype(jnp.float32)
            xb = x_ref[pl.ds(r0, BS), :]
            h = lax.dot_general(xb, w1_ref[0], (((1,), (0,)), ((), ())),
                                preferred_element_type=jnp.float32)
            h = jnp.maximum(h + b1_ref[0, 0][None, :], 0.0)
            contrib = lax.dot_general(h, w2_ref[0], (((1,), (0,)), ((), ())),
                                      preferred_element_type=jnp.float32)
            isk0 = (k == 0).astype(jnp.float32)
            contrib = (contrib + isk0 * b2_ref[0, 0][None, :]) * gate
            out_ref[pl.ds(r0, BS), :] += contrib
            return 0

        lax.fori_loop(0, nsub, sub, 0)


def _grouped_mlp(ts, te, x_sorted, w1, b1, w2, b2):
    grid_spec = pltpu.PrefetchScalarGridSpec(
        num_scalar_prefetch=2,
        grid=(G,),
        in_specs=[
            pl.BlockSpec((N, DI), lambda g, s0, e0: (0, 0)),
            pl.BlockSpec((1, DI, TH),
                         lambda g, s0, e0: (g // K, 0, g % K)),
            pl.BlockSpec((1, 1, TH),
                         lambda g, s0, e0: (g, 0, 0)),
            pl.BlockSpec((1, TH, DO),
                         lambda g, s0, e0: (g // K, g % K, 0)),
            pl.BlockSpec((1, 1, DO),
                         lambda g, s0, e0: (g // K, 0, 0)),
        ],
        out_specs=pl.BlockSpec((N, DO), lambda g, s0, e0: (0, 0)),
    )
    return pl.pallas_call(
        _mlp_body,
        grid_spec=grid_spec,
        out_shape=jax.ShapeDtypeStruct((N, DO), jnp.float32),
        compiler_params=pltpu.CompilerParams(
            dimension_semantics=("arbitrary",)),
    )(ts, te, x_sorted, w1,
      b1.reshape(C * K, 1, TH), w2, b2.reshape(C, 1, DO))


def kernel(x, cat_ids, W1, b1, W2, b2):
    cat32 = cat_ids.astype(jnp.int32)
    onehot = (cat32[:, None] == jnp.arange(C, dtype=jnp.int32)[None, :]
              ).astype(jnp.int32)
    ranks = jnp.cumsum(onehot, axis=0)
    counts = ranks[-1]
    starts = (jnp.cumsum(counts) - counts).astype(jnp.int32)
    ends = (starts + counts).astype(jnp.int32)
    rank_i = jnp.take_along_axis(ranks, cat32[:, None], axis=1)[:, 0] - 1
    position = (starts[cat32] + rank_i).astype(jnp.int32)

    x_sorted = _sc_scatter(x, position, N)
    out_sorted = _grouped_mlp(starts, ends, x_sorted, W1, b1, W2, b2)
    return _sc_gather(out_sorted, position)


# speedup vs baseline: 1.0448x; 1.0448x over previous
import functools

import jax
import jax.numpy as jnp
from jax import lax
from jax.experimental import pallas as pl
from jax.experimental.pallas import tpu as pltpu
from jax.experimental.pallas import tpu_sc as plsc

C = 16
DI = 1024
DH = 4096
DO = 1024
N = 4096
BS = 128
TH = 1024
K = DH // TH
G = C * K

_NW = 32
_CH = 64

def _sc_mesh():
    return plsc.VectorSubcoreMesh(core_axis_name="c", subcore_axis_name="s")


def _sc_gather(table, idx):
    v, d = table.shape
    bn = idx.shape[0]
    per_w = bn // _NW
    nch = per_w // _CH

    @functools.partial(
        pl.kernel, mesh=_sc_mesh(),
        out_type=jax.ShapeDtypeStruct((bn, d), table.dtype),
        scratch_types=[pltpu.VMEM((_CH,), jnp.int32),
                       pltpu.VMEM((_CH, d), table.dtype),
                       pltpu.SemaphoreType.DMA])
    def k(table_hbm, idx_hbm, out_hbm, idx_v, rows_v, sem):
        wid = lax.axis_index("s") * 2 + lax.axis_index("c")

        @pl.loop(0, nch)
        def _(i):
            base = wid * per_w + i * _CH
            pltpu.sync_copy(idx_hbm.at[pl.ds(base, _CH)], idx_v)
            pltpu.async_copy(table_hbm.at[idx_v], rows_v, sem).wait()
            pltpu.sync_copy(rows_v, out_hbm.at[pl.ds(base, _CH)])

    return k(table, idx)


def _sc_scatter(rows, idx, v):
    bn, d = rows.shape
    per_w = bn // _NW
    nch = per_w // _CH

    @functools.partial(
        pl.kernel, mesh=_sc_mesh(),
        out_type=jax.ShapeDtypeStruct((v, d), rows.dtype),
        scratch_types=[pltpu.VMEM((_CH,), jnp.int32),
                       pltpu.VMEM((_CH, d), rows.dtype),
                       pltpu.SemaphoreType.DMA])
    def k(rows_hbm, idx_hbm, out_hbm, idx_v, rows_v, sem):
        wid = lax.axis_index("s") * 2 + lax.axis_index("c")

        @pl.loop(0, nch)
        def _(i):
            base = wid * per_w + i * _CH
            pltpu.sync_copy(idx_hbm.at[pl.ds(base, _CH)], idx_v)
            pltpu.sync_copy(rows_hbm.at[pl.ds(base, _CH)], rows_v)
            pltpu.async_copy(rows_v, out_hbm.at[idx_v], sem).wait()

    return k(rows, idx)


def _mlp_body(ts_r, te_r,
              x_ref, w1_ref, b1_ref, w2_ref, b2_ref, out_ref):
    g = pl.program_id(0)
    c = g // K
    k = g - c * K

    @pl.when(g == 0)
    def _init():
        out_ref[...] = jnp.zeros_like(out_ref)

    s = ts_r[c]
    e = te_r[c]

    @pl.when(e > s)
    def _work():
        a0 = (s // 8) * 8
        nsub = (e - a0 + BS - 1) // BS

        def sub(j, _):
            r0l = a0 + j * BS
            r0 = jnp.minimum(r0l, N - BS)
            rows = r0 + lax.broadcasted_iota(jnp.int32, (BS, 1), 0)
            gate = ((rows >= jnp.maximum(s, r0l)) & (rows < e)
                    ).astype(jnp.float32)
            xb = x_ref[pl.ds(r0, BS), :]
            h = lax.dot_general(xb, w1_ref[0], (((1,), (0,)), ((), ())),
                                preferred_element_type=jnp.float32)
            h = jnp.maximum(h + b1_ref[0, 0][None, :], 0.0)
            contrib = lax.dot_general(h, w2_ref[0], (((1,), (0,)), ((), ())),
                                      preferred_element_type=jnp.float32)
            isk0 = (k == 0).astype(jnp.float32)
            contrib = (contrib + isk0 * b2_ref[0, 0][None, :]) * gate
            out_ref[pl.ds(r0, BS), :] += contrib
            return 0

        lax.fori_loop(0, nsub, sub, 0)


def _grouped_mlp(ts, te, x_sorted, w1, b1, w2, b2):
    grid_spec = pltpu.PrefetchScalarGridSpec(
        num_scalar_prefetch=2,
        grid=(G,),
        in_specs=[
            pl.BlockSpec((N, DI), lambda g, s0, e0: (0, 0)),
            pl.BlockSpec((1, DI, TH),
                         lambda g, s0, e0: (g // K, 0, g % K)),
            pl.BlockSpec((1, 1, TH),
                         lambda g, s0, e0: (g, 0, 0)),
            pl.BlockSpec((1, TH, DO),
                         lambda g, s0, e0: (g // K, g % K, 0)),
            pl.BlockSpec((1, 1, DO),
                         lambda g, s0, e0: (g // K, 0, 0)),
        ],
        out_specs=pl.BlockSpec((N, DO), lambda g, s0, e0: (0, 0)),
    )
    return pl.pallas_call(
        _mlp_body,
        grid_spec=grid_spec,
        out_shape=jax.ShapeDtypeStruct((N, DO), jnp.float32),
        compiler_params=pltpu.CompilerParams(
            dimension_semantics=("arbitrary",)),
    )(ts, te, x_sorted, w1,
      b1.reshape(C * K, 1, TH), w2, b2.reshape(C, 1, DO))


def kernel(x, cat_ids, W1, b1, W2, b2):
    cat32 = cat_ids.astype(jnp.int32)
    order = jnp.argsort(cat32).astype(jnp.int32)
    sorted_cat = jnp.take(cat32, order)
    cats = jnp.arange(C, dtype=jnp.int32)
    starts = jnp.searchsorted(sorted_cat, cats, side="left").astype(jnp.int32)
    ends = jnp.searchsorted(sorted_cat, cats, side="right").astype(jnp.int32)

    x_sorted = _sc_gather(x, order)
    out_sorted = _grouped_mlp(starts, ends, x_sorted, W1, b1, W2, b2)
    return _sc_scatter(out_sorted, order, N)


# speedup vs baseline: 1.0489x; 1.0039x over previous
import functools

import jax
import jax.numpy as jnp
from jax import lax
from jax.experimental import pallas as pl
from jax.experimental.pallas import tpu as pltpu
from jax.experimental.pallas import tpu_sc as plsc

C = 16
DI = 1024
DH = 4096
DO = 1024
N = 4096
BS = 256
TH = 1024
K = DH // TH
G = C * K

_NW = 32
_CH = 64

def _sc_mesh():
    return plsc.VectorSubcoreMesh(core_axis_name="c", subcore_axis_name="s")


def _sc_gather(table, idx):
    v, d = table.shape
    bn = idx.shape[0]
    per_w = bn // _NW
    nch = per_w // _CH

    @functools.partial(
        pl.kernel, mesh=_sc_mesh(),
        out_type=jax.ShapeDtypeStruct((bn, d), table.dtype),
        scratch_types=[pltpu.VMEM((_CH,), jnp.int32),
                       pltpu.VMEM((_CH, d), table.dtype),
                       pltpu.SemaphoreType.DMA])
    def k(table_hbm, idx_hbm, out_hbm, idx_v, rows_v, sem):
        wid = lax.axis_index("s") * 2 + lax.axis_index("c")

        @pl.loop(0, nch)
        def _(i):
            base = wid * per_w + i * _CH
            pltpu.sync_copy(idx_hbm.at[pl.ds(base, _CH)], idx_v)
            pltpu.async_copy(table_hbm.at[idx_v], rows_v, sem).wait()
            pltpu.sync_copy(rows_v, out_hbm.at[pl.ds(base, _CH)])

    return k(table, idx)


def _sc_scatter(rows, idx, v):
    bn, d = rows.shape
    per_w = bn // _NW
    nch = per_w // _CH

    @functools.partial(
        pl.kernel, mesh=_sc_mesh(),
        out_type=jax.ShapeDtypeStruct((v, d), rows.dtype),
        scratch_types=[pltpu.VMEM((_CH,), jnp.int32),
                       pltpu.VMEM((_CH, d), rows.dtype),
                       pltpu.SemaphoreType.DMA])
    def k(rows_hbm, idx_hbm, out_hbm, idx_v, rows_v, sem):
        wid = lax.axis_index("s") * 2 + lax.axis_index("c")

        @pl.loop(0, nch)
        def _(i):
            base = wid * per_w + i * _CH
            pltpu.sync_copy(idx_hbm.at[pl.ds(base, _CH)], idx_v)
            pltpu.sync_copy(rows_hbm.at[pl.ds(base, _CH)], rows_v)
            pltpu.async_copy(rows_v, out_hbm.at[idx_v], sem).wait()

    return k(rows, idx)


def _mlp_body(ts_r, te_r,
              x_ref, w1_ref, b1_ref, w2_ref, b2_ref, out_ref):
    g = pl.program_id(0)
    c = g // K
    k = g - c * K

    @pl.when(g == 0)
    def _init():
        out_ref[...] = jnp.zeros_like(out_ref)

    s = ts_r[c]
    e = te_r[c]

    @pl.when(e > s)
    def _work():
        a0 = (s // 8) * 8
        nsub = (e - a0 + BS - 1) // BS

        def sub(j, _):
            r0l = a0 + j * BS
            r0 = jnp.minimum(r0l, N - BS)
            rows = r0 + lax.broadcasted_iota(jnp.int32, (BS, 1), 0)
            gate = ((rows >= jnp.maximum(s, r0l)) & (rows < e)
                    ).astype(jnp.float32)
            xb = x_ref[pl.ds(r0, BS), :]
            h = lax.dot_general(xb, w1_ref[0], (((1,), (0,)), ((), ())),
                                preferred_element_type=jnp.float32)
            h = jnp.maximum(h + b1_ref[0, 0][None, :], 0.0)
            contrib = lax.dot_general(h, w2_ref[0], (((1,), (0,)), ((), ())),
                                      preferred_element_type=jnp.float32)
            isk0 = (k == 0).astype(jnp.float32)
            contrib = (contrib + isk0 * b2_ref[0, 0][None, :]) * gate
            out_ref[pl.ds(r0, BS), :] += contrib
            return 0

        lax.fori_loop(0, nsub, sub, 0)


def _grouped_mlp(ts, te, x_sorted, w1, b1, w2, b2):
    grid_spec = pltpu.PrefetchScalarGridSpec(
        num_scalar_prefetch=2,
        grid=(G,),
        in_specs=[
            pl.BlockSpec((N, DI), lambda g, s0, e0: (0, 0)),
            pl.BlockSpec((1, DI, TH),
                         lambda g, s0, e0: (g // K, 0, g % K)),
            pl.BlockSpec((1, 1, TH),
                         lambda g, s0, e0: (g, 0, 0)),
            pl.BlockSpec((1, TH, DO),
                         lambda g, s0, e0: (g // K, g % K, 0)),
            pl.BlockSpec((1, 1, DO),
                         lambda g, s0, e0: (g // K, 0, 0)),
        ],
        out_specs=pl.BlockSpec((N, DO), lambda g, s0, e0: (0, 0)),
    )
    return pl.pallas_call(
        _mlp_body,
        grid_spec=grid_spec,
        out_shape=jax.ShapeDtypeStruct((N, DO), jnp.float32),
        compiler_params=pltpu.CompilerParams(
            dimension_semantics=("arbitrary",)),
    )(ts, te, x_sorted, w1,
      b1.reshape(C * K, 1, TH), w2, b2.reshape(C, 1, DO))


def kernel(x, cat_ids, W1, b1, W2, b2):
    cat32 = cat_ids.astype(jnp.int32)
    order = jnp.argsort(cat32).astype(jnp.int32)
    sorted_cat = jnp.take(cat32, order)
    cats = jnp.arange(C, dtype=jnp.int32)
    starts = jnp.searchsorted(sorted_cat, cats, side="left").astype(jnp.int32)
    ends = jnp.searchsorted(sorted_cat, cats, side="right").astype(jnp.int32)

    x_sorted = _sc_gather(x, order)
    out_sorted = _grouped_mlp(starts, ends, x_sorted, W1, b1, W2, b2)
    return _sc_scatter(out_sorted, order, N)
